# baseline (speedup 1.0000x reference)
import functools

import jax
import jax.numpy as jnp
from jax import lax
from jax.experimental import pallas as pl
from jax.experimental.pallas import tpu as pltpu
from jax.experimental.pallas import tpu_sc as plsc

NUM_CORES = 2
NUM_SUBCORES = 16
NUM_WORKERS = NUM_CORES * NUM_SUBCORES
L = 16
BLK = 128
PSTRIDE = 129

_TC_TILED = pltpu.CompilerParams(use_tc_tiling_on_sc=True,
                                 needs_layout_passes=False)
_LINEAR = pltpu.CompilerParams(use_tc_tiling_on_sc=False)


@functools.lru_cache(maxsize=None)
def _make_format(v: int, d: int):
    nfull = v // BLK
    rem = v - nfull * BLK
    per_w = (nfull + (1 if rem else 0) + NUM_WORKERS - 1) // NUM_WORKERS
    mesh = plsc.VectorSubcoreMesh(core_axis_name="c", subcore_axis_name="s")

    scratch = [
        pltpu.VMEM((d, BLK), jnp.float32),
        pltpu.VMEM((d, BLK), jnp.float32),
        pltpu.VMEM((BLK // 2, 2 * d), jnp.float32),
        pltpu.VMEM((BLK // 2, 2 * d), jnp.float32),
        pltpu.VMEM((PSTRIDE * 64,), jnp.float32),
        pltpu.SemaphoreType.DMA,
        pltpu.SemaphoreType.DMA,
        pltpu.SemaphoreType.DMA,
        pltpu.SemaphoreType.DMA,
    ]
    if rem:
        scratch += [
            pltpu.VMEM((d, rem), jnp.float32),
            pltpu.VMEM((rem // 2, 2 * d), jnp.float32),
            pltpu.SemaphoreType.DMA,
        ]

    @functools.partial(
        pl.kernel,
        out_type=jax.ShapeDtypeStruct((v // 2, 2 * d), jnp.float32),
        mesh=mesh,
        compiler_params=_TC_TILED,
        scratch_types=scratch,
    )
    def k1(tt_hbm, t2_hbm, tin0, tin1, tout0, tout1, stage,
           isem0, isem1, osem0, osem1, *rest):
        wid = lax.axis_index("s") * NUM_CORES + lax.axis_index("c")
        lane = lax.iota(jnp.int32, L)
        gvecs = [(lane + q * L) * PSTRIDE for q in range(d // L)]

        def transpose(src, dst, ncols):
            def drow(dd, c):
                for q in range(ncols // L):
                    stage[pl.ds(dd * PSTRIDE + q * L, L)] = \
                        src[dd, pl.ds(q * L, L)]
                return c
            lax.fori_loop(0, d, drow, 0, unroll=8)

            def vrow(r, c):
                for q in range(d // L):
                    dst[r, pl.ds(q * L, L)] = \
                        plsc.load_gather(stage, [gvecs[q] + 2 * r])
                    dst[r, pl.ds(d + q * L, L)] = \
                        plsc.load_gather(stage, [gvecs[q] + (2 * r + 1)])
                return c
            lax.fori_loop(0, ncols // 2, vrow, 0, unroll=8)

        def fire_in(i, tin, isem):
            b = i * NUM_WORKERS + wid

            @pl.when(b < nfull)
            def _():
                v0 = pl.multiple_of(b * BLK, BLK)
                pltpu.async_copy(tt_hbm.at[:, pl.ds(v0, BLK)], tin, isem)

        def wait_in(i, tin, isem):
            b = i * NUM_WORKERS + wid

            @pl.when(b < nfull)
            def _():
                v0 = pl.multiple_of(b * BLK, BLK)
                pltpu.make_async_copy(
                    tt_hbm.at[:, pl.ds(v0, BLK)], tin, isem).wait()

        def fire_out(i, tout, osem):
            b = i * NUM_WORKERS + wid

            @pl.when(b < nfull)
            def _():
                r0 = pl.multiple_of(b * (BLK // 2), BLK // 2)
                pltpu.async_copy(tout, t2_hbm.at[pl.ds(r0, BLK // 2)], osem)

        def wait_out(i, tout, osem):
            b = i * NUM_WORKERS + wid

            @pl.when(b < nfull)
            def _():
                r0 = pl.multiple_of(b * (BLK // 2), BLK // 2)
                pltpu.make_async_copy(
                    tout, t2_hbm.at[pl.ds(r0, BLK // 2)], osem).wait()

        bufs = ((tin0, isem0, tout0, osem0), (tin1, isem1, tout1, osem1))
        fire_in(0, tin0, isem0)

        def pair_body(i2, carry):
            for par in range(2):
                i = 2 * i2 + par
                tin, isem, tout, osem = bufs[par]
                ntin, nisem, _, _ = bufs[1 - par]
                wait_in(i, tin, isem)
                fire_in(i + 1, ntin, nisem)

                @pl.when(i >= 2)
                def _():
                    wait_out(i - 2, tout, osem)

                b = i * NUM_WORKERS + wid

                @pl.when(b < nfull)
                def _():
                    transpose(tin, tout, BLK)
                fire_out(i, tout, osem)
            return carry

        npairs = (per_w + 1) // 2
        lax.fori_loop(0, npairs, pair_body, 0)
        wait_out(2 * npairs - 2, tout0, osem0)
        wait_out(2 * npairs - 1, tout1, osem1)

        if rem:
            tin_r, tout_r, rsem = rest

            @pl.when(wid == nfull % NUM_WORKERS)
            def _():
                v0 = nfull * BLK
                pltpu.async_copy(tt_hbm.at[:, pl.ds(v0, rem)], tin_r,
                                 rsem).wait()
                transpose(tin_r, tout_r, rem)
                pltpu.async_copy(tout_r,
                                 t2_hbm.at[pl.ds(v0 // 2, rem // 2)],
                                 rsem).wait()

    return k1


@functools.lru_cache(maxsize=None)
def _make_gather(total: int, v: int, d: int):
    chunks_per_w = total // BLK // NUM_WORKERS
    assert chunks_per_w % 2 == 0
    mesh = plsc.VectorSubcoreMesh(core_axis_name="c", subcore_axis_name="s")

    @functools.partial(
        pl.kernel,
        out_type=jax.ShapeDtypeStruct((total, 2 * d), jnp.float32),
        mesh=mesh,
        compiler_params=_LINEAR,
        scratch_types=(
            [pltpu.VMEM((chunks_per_w, BLK), jnp.int32)]
            + [pltpu.VMEM((BLK, d), jnp.float32)] * 4
            + [pltpu.SemaphoreType.DMA] * 8
        ),
    )
    def k2(t_hbm, idx_hbm, out_hbm, idx_v, b0, b1, b2, b3, *sems):
        wid = lax.axis_index("s") * NUM_CORES + lax.axis_index("c")
        base = wid * chunks_per_w
        pltpu.sync_copy(idx_hbm.at[pl.ds(base, chunks_per_w)], idx_v)
        bufs = (b0, b1, b2, b3)
        gs = sems[:4]
        ws = sems[4:]

        def _gather(j, par, fire):
            f = pltpu.async_copy if fire else pltpu.make_async_copy
            return f(t_hbm.at[idx_v.at[j]], bufs[par], gs[par])

        def _wb(j, par, fire):
            f = pltpu.async_copy if fire else pltpu.make_async_copy
            dst = pl.multiple_of((base + j) * BLK, BLK)
            return f(bufs[par], out_hbm.at[pl.ds(dst, BLK), pl.ds(0, d)],
                     ws[par])

        for par in range(3):
            _gather(par, par, True)

        last = chunks_per_w - 1

        def body(p, carry):
            for par in range(4):
                j = 4 * p + par
                _gather(j, par, False).wait()
                _wb(j, par, True)
                prev = (par - 1) % 4
                if par == 0:
                    @pl.when(p > 0)
                    def _():
                        _wb(j - 1, prev, False).wait()

                    @pl.when(j + 3 <= last)
                    def _():
                        _gather(j + 3, prev, True)
                else:
                    _wb(j - 1, prev, False).wait()

                    @pl.when(j + 3 <= last)
                    def _():
                        _gather(j + 3, prev, True)
            return carry

        lax.fori_loop(0, chunks_per_w // 4, body, 0)
        _wb(last, last % 4, False).wait()

    return k2


def kernel(stacks, table):
    batch, hist = stacks.shape
    v, d = table.shape
    total = batch * hist
    t2 = _make_format(v, d)(table.T)
    t1m = t2.reshape(v, d)
    idx = stacks.reshape(total // BLK, BLK).astype(jnp.int32)
    out = _make_gather(total, v, d)(t1m, idx)
    return out[:, :d].reshape(batch, hist, d)


# speedup vs baseline: 1.3588x; 1.3588x over previous
import functools

import jax
import jax.numpy as jnp
from jax import lax
from jax.experimental import pallas as pl
from jax.experimental.pallas import tpu as pltpu
from jax.experimental.pallas import tpu_sc as plsc

NUM_CORES = 2
NUM_SUBCORES = 16
NUM_WORKERS = NUM_CORES * NUM_SUBCORES
BLK = 128
NBUF = 4

_LINEAR = pltpu.CompilerParams(use_tc_tiling_on_sc=False)


@functools.lru_cache(maxsize=None)
def _make_gather(total: int, v: int, d: int):
    chunks_per_w = total // BLK // NUM_WORKERS
    assert chunks_per_w % NBUF == 0
    mesh = plsc.VectorSubcoreMesh(core_axis_name="c", subcore_axis_name="s")

    @functools.partial(
        pl.kernel,
        out_type=jax.ShapeDtypeStruct((total, 2 * d), jnp.float32),
        mesh=mesh,
        compiler_params=_LINEAR,
        scratch_types=(
            [pltpu.VMEM((chunks_per_w, BLK), jnp.int32)]
            + [pltpu.VMEM((BLK, d), jnp.float32)] * NBUF
            + [pltpu.SemaphoreType.DMA] * (2 * NBUF)
        ),
    )
    def k2(t_hbm, idx_hbm, out_hbm, idx_v, *bufs_sems):
        wid = lax.axis_index("s") * NUM_CORES + lax.axis_index("c")
        base = wid * chunks_per_w
        pltpu.sync_copy(idx_hbm.at[pl.ds(base, chunks_per_w)], idx_v)
        bufs = bufs_sems[:NBUF]
        gs = bufs_sems[NBUF:2 * NBUF]
        ws = bufs_sems[2 * NBUF:]

        def _gather(j, par, fire):
            f = pltpu.async_copy if fire else pltpu.make_async_copy
            return f(t_hbm.at[idx_v.at[j]], bufs[par], gs[par])

        def _wb(j, par, fire):
            f = pltpu.async_copy if fire else pltpu.make_async_copy
            dst = pl.multiple_of((base + j) * BLK, BLK)
            return f(bufs[par], out_hbm.at[pl.ds(dst, BLK), pl.ds(0, d)],
                     ws[par])

        for par in range(NBUF - 1):
            _gather(par, par, True)

        last = chunks_per_w - 1

        def body(p, carry):
            for par in range(NBUF):
                j = NBUF * p + par
                _gather(j, par, False).wait()
                _wb(j, par, True)
                prev = (par - 1) % NBUF
                if par == 0:
                    @pl.when(p > 0)
                    def _():
                        _wb(j - 1, prev, False).wait()

                    @pl.when(j + NBUF - 1 <= last)
                    def _():
                        _gather(j + NBUF - 1, prev, True)
                else:
                    _wb(j - 1, prev, False).wait()

                    @pl.when(j + NBUF - 1 <= last)
                    def _():
                        _gather(j + NBUF - 1, prev, True)
            return carry

        lax.fori_loop(0, chunks_per_w // NBUF, body, 0)
        _wb(last, last % NBUF, False).wait()

    return k2


def kernel(stacks, table):
    batch, hist = stacks.shape
    v, d = table.shape
    total = batch * hist
    idx = stacks.reshape(total // BLK, BLK).astype(jnp.int32)
    out = _make_gather(total, v, d)(table, idx)
    return out[:, :d].reshape(batch, hist, d)
